# baseline (speedup 1.0000x reference)
import jax
import jax.numpy as jnp
from jax import lax
from jax.experimental import pallas as pl
from jax.experimental.pallas import tpu as pltpu
from jax.experimental.pallas import tpu_sc as plsc

HIDDEN = 128
N_NODES = 10000
N_EDGES = 320000

NC = 2
NS = 16
NW = NC * NS
E_PER_TILE = N_EDGES // NW
CHUNK = 40
NCHUNKS = E_PER_TILE // CHUNK
U = 5
N_PAD = 10240
ROWS_PER_TILE = N_PAD // NS


SDW = 16


def _sc_body(x_hbm, src_hbm, dst_hbm, sd_hbm, z128_hbm, zsd_hbm,
             agg_out, sd_out,
             src_v, dst_v, sd_v, rows_v, acc_sh, sd_sh, gsems, ssems, dsems):
  c = lax.axis_index("c")
  s = lax.axis_index("s")
  wid = c * NS + s
  ebase = wid * E_PER_TILE

  rbase = s * ROWS_PER_TILE
  pltpu.sync_copy(z128_hbm.at[pl.ds(rbase, ROWS_PER_TILE)],
                  acc_sh.at[pl.ds(rbase, ROWS_PER_TILE)])
  pltpu.sync_copy(zsd_hbm.at[pl.ds(rbase, ROWS_PER_TILE)],
                  sd_sh.at[pl.ds(rbase, ROWS_PER_TILE)])
  plsc.subcore_barrier()

  def block_body(i, carry):
    for k in range(U):
      off = ebase + (i * U + k) * CHUNK
      pltpu.sync_copy(src_hbm.at[pl.ds(off, CHUNK)], src_v.at[k])
      pltpu.sync_copy(dst_hbm.at[pl.ds(off, CHUNK)], dst_v.at[k])
      pltpu.sync_copy(sd_hbm.at[pl.ds(off, CHUNK)], sd_v.at[k])
    gd = [pltpu.async_copy(x_hbm.at[src_v.at[k]], rows_v.at[k], gsems.at[k])
          for k in range(U)]
    sc = []
    for k in range(U):
      gd[k].wait()
      sc.append(pltpu.async_copy(rows_v.at[k], acc_sh.at[dst_v.at[k]],
                                 ssems.at[k], add=True))
      sc.append(pltpu.async_copy(sd_v.at[k], sd_sh.at[dst_v.at[k]],
                                 dsems.at[k], add=True))
    for d in sc:
      d.wait()
    return carry

  lax.fori_loop(0, NCHUNKS // U, block_body, 0)
  plsc.subcore_barrier()

  obase = c * N_PAD + rbase
  pltpu.sync_copy(acc_sh.at[pl.ds(rbase, ROWS_PER_TILE)],
                  agg_out.at[pl.ds(obase, ROWS_PER_TILE)])
  pltpu.sync_copy(sd_sh.at[pl.ds(rbase, ROWS_PER_TILE)],
                  sd_out.at[pl.ds(obase, ROWS_PER_TILE)])


_sc_aggregate = pl.kernel(
    _sc_body,
    out_type=(
        jax.ShapeDtypeStruct((NC * N_PAD, HIDDEN), jnp.float32),
        jax.ShapeDtypeStruct((NC * N_PAD, SDW), jnp.float32),
    ),
    mesh=plsc.VectorSubcoreMesh(core_axis_name="c", subcore_axis_name="s",
                                num_cores=NC, num_subcores=NS),
    scratch_types=[
        pltpu.VMEM((U, CHUNK), jnp.int32),
        pltpu.VMEM((U, CHUNK), jnp.int32),
        pltpu.VMEM((U, CHUNK, SDW), jnp.float32),
        pltpu.VMEM((U, CHUNK, HIDDEN), jnp.float32),
        pltpu.VMEM_SHARED((N_PAD, HIDDEN), jnp.float32),
        pltpu.VMEM_SHARED((N_PAD, SDW), jnp.float32),
        pltpu.SemaphoreType.DMA((U,)),
        pltpu.SemaphoreType.DMA((U,)),
        pltpu.SemaphoreType.DMA((U,)),
    ],
    compiler_params=pltpu.CompilerParams(use_tc_tiling_on_sc=False),
)


ROW_BLK = 1000


def _mlp_body(p0, p1, sd0, sd1, x, We, be, W1, b1, W2, b2, o):
  sd = sd0[...] + sd1[...]
  pre = (p0[...] + p1[...] + x[...]
         + sd[:, 0:1] * We[...] + sd[:, 1:2] * be[...])
  h = jnp.maximum(
      jnp.dot(pre, W1[...], preferred_element_type=jnp.float32) + b1[...], 0.0)
  o[...] = jnp.dot(h, W2[...], preferred_element_type=jnp.float32) + b2[...]


def _mlp_call(p0, p1, sd0, sd1, x, We, be, W1, b1, W2, b2):
  grid = (N_NODES // ROW_BLK,)
  row = lambda i: (i, 0)
  fix = lambda i: (0, 0)
  return pl.pallas_call(
      _mlp_body,
      grid=grid,
      in_specs=[
          pl.BlockSpec((ROW_BLK, HIDDEN), row),
          pl.BlockSpec((ROW_BLK, HIDDEN), row),
          pl.BlockSpec((ROW_BLK, SDW), row),
          pl.BlockSpec((ROW_BLK, SDW), row),
          pl.BlockSpec((ROW_BLK, HIDDEN), row),
          pl.BlockSpec((1, HIDDEN), fix),
          pl.BlockSpec((1, HIDDEN), fix),
          pl.BlockSpec((HIDDEN, HIDDEN), fix),
          pl.BlockSpec((1, HIDDEN), fix),
          pl.BlockSpec((HIDDEN, HIDDEN), fix),
          pl.BlockSpec((1, HIDDEN), fix),
      ],
      out_specs=pl.BlockSpec((ROW_BLK, HIDDEN), row),
      out_shape=jax.ShapeDtypeStruct((N_NODES, HIDDEN), jnp.float32),
  )(p0, p1, sd0, sd1, x, We, be, W1, b1, W2, b2)


def kernel(x, edge_index, edge_weight, We, be, W1, b1, W2, b2):
  src = edge_index[0].astype(jnp.int32)
  dst = edge_index[1].astype(jnp.int32)
  ew = edge_weight.astype(jnp.float32)
  sd = (jnp.zeros((N_EDGES, SDW), jnp.float32)
        .at[:, 0].set(ew).at[:, 1].set(1.0))
  z128 = jnp.zeros((N_PAD, HIDDEN), jnp.float32)
  zsd = jnp.zeros((N_PAD, SDW), jnp.float32)
  agg, sdp = _sc_aggregate(x, src, dst, sd, z128, zsd)
  return _mlp_call(agg[:N_NODES], agg[N_PAD:N_PAD + N_NODES],
                   sdp[:N_NODES], sdp[N_PAD:N_PAD + N_NODES],
                   x, We, be.reshape(1, HIDDEN), W1, b1.reshape(1, HIDDEN),
                   W2, b2.reshape(1, HIDDEN))


# speedup vs baseline: 3.4129x; 3.4129x over previous
import jax
import jax.numpy as jnp
from jax import lax
from jax.experimental import pallas as pl
from jax.experimental.pallas import tpu as pltpu
from jax.experimental.pallas import tpu_sc as plsc

HIDDEN = 128
N_NODES = 10000
N_EDGES = 320000

NC = 2
NS = 16
NW = NC * NS
E_PER_TILE = N_EDGES // NW
CHUNK = 80
NCHUNKS = E_PER_TILE // CHUNK
ROWS_PER_TILE = N_NODES // NS
LANES = 16


def _sc_body(x_hbm, src_hbm, dst_hbm, ew_hbm, z128_hbm, z1_hbm,
             agg_out, sd_out,
             src_v, dst_v, ew_v, rows_v, s_part, d_part, acc_sh, sem):
  c = lax.axis_index("c")
  s = lax.axis_index("s")
  wid = c * NS + s
  ebase = wid * E_PER_TILE

  rbase = s * ROWS_PER_TILE
  pltpu.sync_copy(z128_hbm.at[pl.ds(rbase, ROWS_PER_TILE)],
                  acc_sh.at[pl.ds(rbase, ROWS_PER_TILE)])
  pltpu.sync_copy(z1_hbm, s_part)
  pltpu.sync_copy(z1_hbm, d_part)
  plsc.subcore_barrier()

  ones16 = jnp.ones((LANES,), jnp.float32)

  def chunk_body(i, carry):
    off = ebase + i * CHUNK
    pltpu.sync_copy(src_hbm.at[pl.ds(off, CHUNK)], src_v)
    pltpu.sync_copy(dst_hbm.at[pl.ds(off, CHUNK)], dst_v)
    pltpu.sync_copy(ew_hbm.at[pl.ds(off, CHUNK)], ew_v)
    gd = pltpu.async_copy(x_hbm.at[src_v], rows_v, sem)
    for g in range(CHUNK // LANES):
      idx = dst_v[pl.ds(g * LANES, LANES)]
      plsc.addupdate_scatter(s_part, [idx], ew_v[pl.ds(g * LANES, LANES)])
      plsc.addupdate_scatter(d_part, [idx], ones16)
    gd.wait()
    pltpu.sync_copy(rows_v, acc_sh.at[dst_v], add=True)
    return carry

  lax.fori_loop(0, NCHUNKS, chunk_body, 0)
  plsc.subcore_barrier()

  pltpu.sync_copy(acc_sh.at[pl.ds(rbase, ROWS_PER_TILE)],
                  agg_out.at[c, pl.ds(rbase, ROWS_PER_TILE)])
  pltpu.sync_copy(s_part, sd_out.at[wid])
  pltpu.sync_copy(d_part, sd_out.at[NW + wid])


_sc_aggregate = pl.kernel(
    _sc_body,
    out_type=(
        jax.ShapeDtypeStruct((NC, N_NODES, HIDDEN), jnp.float32),
        jax.ShapeDtypeStruct((2 * NW, N_NODES), jnp.float32),
    ),
    mesh=plsc.VectorSubcoreMesh(core_axis_name="c", subcore_axis_name="s",
                                num_cores=NC, num_subcores=NS),
    scratch_types=[
        pltpu.VMEM((CHUNK,), jnp.int32),
        pltpu.VMEM((CHUNK,), jnp.int32),
        pltpu.VMEM((CHUNK,), jnp.float32),
        pltpu.VMEM((CHUNK, HIDDEN), jnp.float32),
        pltpu.VMEM((N_NODES,), jnp.float32),
        pltpu.VMEM((N_NODES,), jnp.float32),
        pltpu.VMEM_SHARED((N_NODES, HIDDEN), jnp.float32),
        pltpu.SemaphoreType.DMA,
    ],
    compiler_params=pltpu.CompilerParams(use_tc_tiling_on_sc=False,
                                         needs_layout_passes=False),
)


ROW_BLK = 1024


def _mlp_body(p0, p1, sdp, sdm, x, W1, b1, W2, b2, o):
  corr = lax.dot_general(sdp[...], sdm[...], (((0,), (0,)), ((), ())),
                         precision=lax.Precision.HIGHEST,
                         preferred_element_type=jnp.float32)
  pre = p0[...][0] + p1[...][0] + x[...] + corr
  h = jnp.maximum(
      jnp.dot(pre, W1[...], preferred_element_type=jnp.float32) + b1[...], 0.0)
  o[...] = jnp.dot(h, W2[...], preferred_element_type=jnp.float32) + b2[...]


def _mlp_call(agg, sdp, sdm, x, W1, b1, W2, b2):
  nblk = (N_NODES + ROW_BLK - 1) // ROW_BLK
  row = lambda i: (i, 0)
  fix = lambda i: (0, 0)
  return pl.pallas_call(
      _mlp_body,
      grid=(nblk,),
      in_specs=[
          pl.BlockSpec((1, ROW_BLK, HIDDEN), lambda i: (0, i, 0)),
          pl.BlockSpec((1, ROW_BLK, HIDDEN), lambda i: (1, i, 0)),
          pl.BlockSpec((2 * NW, ROW_BLK), lambda i: (0, i)),
          pl.BlockSpec((2 * NW, HIDDEN), fix),
          pl.BlockSpec((ROW_BLK, HIDDEN), row),
          pl.BlockSpec((HIDDEN, HIDDEN), fix),
          pl.BlockSpec((1, HIDDEN), fix),
          pl.BlockSpec((HIDDEN, HIDDEN), fix),
          pl.BlockSpec((1, HIDDEN), fix),
      ],
      out_specs=pl.BlockSpec((ROW_BLK, HIDDEN), row),
      out_shape=jax.ShapeDtypeStruct((N_NODES, HIDDEN), jnp.float32),
  )(agg, agg, sdp, sdm, x, W1, b1, W2, b2)


def kernel(x, edge_index, edge_weight, We, be, W1, b1, W2, b2):
  src = edge_index[0].astype(jnp.int32)
  dst = edge_index[1].astype(jnp.int32)
  ew = edge_weight.astype(jnp.float32)
  z128 = jnp.zeros((N_NODES, HIDDEN), jnp.float32)
  z1 = jnp.zeros((N_NODES,), jnp.float32)
  sdm = jnp.concatenate([jnp.broadcast_to(We.reshape(1, HIDDEN), (NW, HIDDEN)),
                         jnp.broadcast_to(be.reshape(1, HIDDEN), (NW, HIDDEN))])
  agg, sdp = _sc_aggregate(x, src, dst, ew, z128, z1)
  return _mlp_call(agg, sdp, sdm, x, W1, b1.reshape(1, HIDDEN),
                   W2, b2.reshape(1, HIDDEN))


# speedup vs baseline: 3.6299x; 1.0636x over previous
import jax
import jax.numpy as jnp
from jax import lax
from jax.experimental import pallas as pl
from jax.experimental.pallas import tpu as pltpu
from jax.experimental.pallas import tpu_sc as plsc

HIDDEN = 128
N_NODES = 10000
N_EDGES = 320000

NC = 2
NS = 16
NW = NC * NS
E_PER_TILE = N_EDGES // NW
CHUNK = 80
NCHUNKS = E_PER_TILE // CHUNK
ROWS_PER_TILE = N_NODES // NS
LANES = 16


def _sc_body(x_hbm, src_hbm, dst_hbm, ew_hbm, z128_hbm, z1_hbm,
             agg_out, sd_out,
             src_v, dst_v, ew_v, rows_v, s_part, d_part, acc_sh, gsems, ssems):
  c = lax.axis_index("c")
  s = lax.axis_index("s")
  wid = c * NS + s
  ebase = wid * E_PER_TILE

  rbase = s * ROWS_PER_TILE
  pltpu.sync_copy(z128_hbm.at[pl.ds(rbase, ROWS_PER_TILE)],
                  acc_sh.at[pl.ds(rbase, ROWS_PER_TILE)])
  pltpu.sync_copy(z1_hbm, s_part)
  pltpu.sync_copy(z1_hbm, d_part)
  plsc.subcore_barrier()

  ones16 = jnp.ones((LANES,), jnp.float32)

  def do_chunk(i, k, scatter_sem):
    off = ebase + i * CHUNK
    pltpu.sync_copy(src_hbm.at[pl.ds(off, CHUNK)], src_v.at[k])
    pltpu.sync_copy(dst_hbm.at[pl.ds(off, CHUNK)], dst_v.at[k])
    pltpu.sync_copy(ew_hbm.at[pl.ds(off, CHUNK)], ew_v.at[k])
    gd = pltpu.async_copy(x_hbm.at[src_v.at[k]], rows_v.at[k], gsems.at[k])
    for g in range(CHUNK // LANES):
      idx = dst_v[k, pl.ds(g * LANES, LANES)]
      plsc.addupdate_scatter(s_part, [idx], ew_v[k, pl.ds(g * LANES, LANES)])
      plsc.addupdate_scatter(d_part, [idx], ones16)
    gd.wait()
    return pltpu.async_copy(rows_v.at[k], acc_sh.at[dst_v.at[k]],
                            scatter_sem, add=True)

  def pair_body(i, carry):
    s0 = do_chunk(2 * i, 0, ssems.at[0])
    s1 = do_chunk(2 * i + 1, 1, ssems.at[1])
    s0.wait()
    s1.wait()
    return carry

  lax.fori_loop(0, NCHUNKS // 2, pair_body, 0)
  do_chunk(NCHUNKS - 1, 0, ssems.at[0]).wait()
  plsc.subcore_barrier()

  pltpu.sync_copy(acc_sh.at[pl.ds(rbase, ROWS_PER_TILE)],
                  agg_out.at[c, pl.ds(rbase, ROWS_PER_TILE)])
  pltpu.sync_copy(s_part, sd_out.at[wid])
  pltpu.sync_copy(d_part, sd_out.at[NW + wid])


_sc_aggregate = pl.kernel(
    _sc_body,
    out_type=(
        jax.ShapeDtypeStruct((NC, N_NODES, HIDDEN), jnp.float32),
        jax.ShapeDtypeStruct((2 * NW, N_NODES), jnp.float32),
    ),
    mesh=plsc.VectorSubcoreMesh(core_axis_name="c", subcore_axis_name="s",
                                num_cores=NC, num_subcores=NS),
    scratch_types=[
        pltpu.VMEM((2, CHUNK), jnp.int32),
        pltpu.VMEM((2, CHUNK), jnp.int32),
        pltpu.VMEM((2, CHUNK), jnp.float32),
        pltpu.VMEM((2, CHUNK, HIDDEN), jnp.float32),
        pltpu.VMEM((N_NODES,), jnp.float32),
        pltpu.VMEM((N_NODES,), jnp.float32),
        pltpu.VMEM_SHARED((N_NODES, HIDDEN), jnp.float32),
        pltpu.SemaphoreType.DMA((2,)),
        pltpu.SemaphoreType.DMA((2,)),
    ],
    compiler_params=pltpu.CompilerParams(use_tc_tiling_on_sc=False,
                                         needs_layout_passes=False),
)


ROW_BLK = 1024


def _mlp_body(p0, p1, sdp, sdm, x, W1, b1, W2, b2, o):
  corr = lax.dot_general(sdp[...], sdm[...], (((0,), (0,)), ((), ())),
                         precision=lax.Precision.HIGHEST,
                         preferred_element_type=jnp.float32)
  pre = p0[...][0] + p1[...][0] + x[...] + corr
  h = jnp.maximum(
      jnp.dot(pre, W1[...], preferred_element_type=jnp.float32) + b1[...], 0.0)
  o[...] = jnp.dot(h, W2[...], preferred_element_type=jnp.float32) + b2[...]


def _mlp_call(agg, sdp, sdm, x, W1, b1, W2, b2):
  nblk = (N_NODES + ROW_BLK - 1) // ROW_BLK
  row = lambda i: (i, 0)
  fix = lambda i: (0, 0)
  return pl.pallas_call(
      _mlp_body,
      grid=(nblk,),
      in_specs=[
          pl.BlockSpec((1, ROW_BLK, HIDDEN), lambda i: (0, i, 0)),
          pl.BlockSpec((1, ROW_BLK, HIDDEN), lambda i: (1, i, 0)),
          pl.BlockSpec((2 * NW, ROW_BLK), lambda i: (0, i)),
          pl.BlockSpec((2 * NW, HIDDEN), fix),
          pl.BlockSpec((ROW_BLK, HIDDEN), row),
          pl.BlockSpec((HIDDEN, HIDDEN), fix),
          pl.BlockSpec((1, HIDDEN), fix),
          pl.BlockSpec((HIDDEN, HIDDEN), fix),
          pl.BlockSpec((1, HIDDEN), fix),
      ],
      out_specs=pl.BlockSpec((ROW_BLK, HIDDEN), row),
      out_shape=jax.ShapeDtypeStruct((N_NODES, HIDDEN), jnp.float32),
  )(agg, agg, sdp, sdm, x, W1, b1, W2, b2)


def kernel(x, edge_index, edge_weight, We, be, W1, b1, W2, b2):
  src = edge_index[0].astype(jnp.int32)
  dst = edge_index[1].astype(jnp.int32)
  ew = edge_weight.astype(jnp.float32)
  z128 = jnp.zeros((N_NODES, HIDDEN), jnp.float32)
  z1 = jnp.zeros((N_NODES,), jnp.float32)
  sdm = jnp.concatenate([jnp.broadcast_to(We.reshape(1, HIDDEN), (NW, HIDDEN)),
                         jnp.broadcast_to(be.reshape(1, HIDDEN), (NW, HIDDEN))])
  agg, sdp = _sc_aggregate(x, src, dst, ew, z128, z1)
  return _mlp_call(agg, sdp, sdm, x, W1, b1.reshape(1, HIDDEN),
                   W2, b2.reshape(1, HIDDEN))


# speedup vs baseline: 5.5291x; 1.5232x over previous
import jax
import jax.numpy as jnp
from jax import lax
from jax.experimental import pallas as pl
from jax.experimental.pallas import tpu as pltpu
from jax.experimental.pallas import tpu_sc as plsc

HIDDEN = 128
N_NODES = 10000
N_EDGES = 320000

NC = 2
NS = 16
NW = NC * NS
E_PER_TILE = N_EDGES // NW
CHUNK = 80
NCHUNKS = E_PER_TILE // CHUNK
SB = 5
ROWS_PER_TILE = N_NODES // NS
LANES = 16


def _sc_body(x_hbm, src_hbm, dst_hbm, ew_hbm, z128_hbm, z1_hbm,
             agg_out, sd_out,
             src_v, dst_v, ew_v, rows_v, s_part, d_part, acc_sh, gsems, ssems):
  c = lax.axis_index("c")
  s = lax.axis_index("s")
  wid = c * NS + s
  ebase = wid * E_PER_TILE

  rbase = s * ROWS_PER_TILE
  pltpu.sync_copy(z128_hbm.at[pl.ds(rbase, ROWS_PER_TILE)],
                  acc_sh.at[pl.ds(rbase, ROWS_PER_TILE)])
  pltpu.sync_copy(z1_hbm, s_part)
  pltpu.sync_copy(z1_hbm, d_part)
  plsc.subcore_barrier()

  ones16 = jnp.ones((LANES,), jnp.float32)
  cbase = wid * NCHUNKS

  def sb_body(i, carry):
    pltpu.sync_copy(src_hbm.at[pl.ds(cbase + i * SB, SB)], src_v)
    pltpu.sync_copy(dst_hbm.at[pl.ds(cbase + i * SB, SB)], dst_v)
    pltpu.sync_copy(ew_hbm.at[pl.ds(cbase + i * SB, SB)], ew_v)
    sdesc = [None, None]
    for k in range(SB):
      b = k % 2
      if sdesc[b] is not None:
        sdesc[b].wait()
      gd = pltpu.async_copy(x_hbm.at[src_v.at[k]], rows_v.at[b], gsems.at[b])
      for g in range(CHUNK // LANES):
        idx = dst_v[k, pl.ds(g * LANES, LANES)]
        plsc.addupdate_scatter(s_part, [idx], ew_v[k, pl.ds(g * LANES, LANES)])
        plsc.addupdate_scatter(d_part, [idx], ones16)
      gd.wait()
      sdesc[b] = pltpu.async_copy(rows_v.at[b], acc_sh.at[dst_v.at[k]],
                                  ssems.at[b], add=True)
    sdesc[0].wait()
    sdesc[1].wait()
    return carry

  lax.fori_loop(0, NCHUNKS // SB, sb_body, 0)
  plsc.subcore_barrier()

  pltpu.sync_copy(acc_sh.at[pl.ds(rbase, ROWS_PER_TILE)],
                  agg_out.at[c, pl.ds(rbase, ROWS_PER_TILE)])
  pltpu.sync_copy(s_part, sd_out.at[wid])
  pltpu.sync_copy(d_part, sd_out.at[NW + wid])


_sc_aggregate = pl.kernel(
    _sc_body,
    out_type=(
        jax.ShapeDtypeStruct((NC, N_NODES, HIDDEN), jnp.float32),
        jax.ShapeDtypeStruct((2 * NW, N_NODES), jnp.float32),
    ),
    mesh=plsc.VectorSubcoreMesh(core_axis_name="c", subcore_axis_name="s",
                                num_cores=NC, num_subcores=NS),
    scratch_types=[
        pltpu.VMEM((SB, CHUNK), jnp.int32),
        pltpu.VMEM((SB, CHUNK), jnp.int32),
        pltpu.VMEM((SB, CHUNK), jnp.float32),
        pltpu.VMEM((2, CHUNK, HIDDEN), jnp.float32),
        pltpu.VMEM((N_NODES,), jnp.float32),
        pltpu.VMEM((N_NODES,), jnp.float32),
        pltpu.VMEM_SHARED((N_NODES, HIDDEN), jnp.float32),
        pltpu.SemaphoreType.DMA((2,)),
        pltpu.SemaphoreType.DMA((2,)),
    ],
    compiler_params=pltpu.CompilerParams(use_tc_tiling_on_sc=False,
                                         needs_layout_passes=False),
)


ROW_BLK = 1024


def _mlp_body(p0, p1, sdp, sdm, x, W1, b1, W2, b2, o):
  corr = lax.dot_general(sdp[...], sdm[...], (((0,), (0,)), ((), ())),
                         precision=lax.Precision.HIGHEST,
                         preferred_element_type=jnp.float32)
  pre = p0[...][0] + p1[...][0] + x[...] + corr
  h = jnp.maximum(
      jnp.dot(pre, W1[...], preferred_element_type=jnp.float32) + b1[...], 0.0)
  o[...] = jnp.dot(h, W2[...], preferred_element_type=jnp.float32) + b2[...]


def _mlp_call(agg, sdp, sdm, x, W1, b1, W2, b2):
  nblk = (N_NODES + ROW_BLK - 1) // ROW_BLK
  row = lambda i: (i, 0)
  fix = lambda i: (0, 0)
  return pl.pallas_call(
      _mlp_body,
      grid=(nblk,),
      in_specs=[
          pl.BlockSpec((1, ROW_BLK, HIDDEN), lambda i: (0, i, 0)),
          pl.BlockSpec((1, ROW_BLK, HIDDEN), lambda i: (1, i, 0)),
          pl.BlockSpec((2 * NW, ROW_BLK), lambda i: (0, i)),
          pl.BlockSpec((2 * NW, HIDDEN), fix),
          pl.BlockSpec((ROW_BLK, HIDDEN), row),
          pl.BlockSpec((HIDDEN, HIDDEN), fix),
          pl.BlockSpec((1, HIDDEN), fix),
          pl.BlockSpec((HIDDEN, HIDDEN), fix),
          pl.BlockSpec((1, HIDDEN), fix),
      ],
      out_specs=pl.BlockSpec((ROW_BLK, HIDDEN), row),
      out_shape=jax.ShapeDtypeStruct((N_NODES, HIDDEN), jnp.float32),
  )(agg, agg, sdp, sdm, x, W1, b1, W2, b2)


def kernel(x, edge_index, edge_weight, We, be, W1, b1, W2, b2):
  src = edge_index[0].astype(jnp.int32).reshape(N_EDGES // CHUNK, CHUNK)
  dst = edge_index[1].astype(jnp.int32).reshape(N_EDGES // CHUNK, CHUNK)
  ew = edge_weight.astype(jnp.float32).reshape(N_EDGES // CHUNK, CHUNK)
  z128 = jnp.zeros((N_NODES, HIDDEN), jnp.float32)
  z1 = jnp.zeros((N_NODES,), jnp.float32)
  sdm = jnp.concatenate([jnp.broadcast_to(We.reshape(1, HIDDEN), (NW, HIDDEN)),
                         jnp.broadcast_to(be.reshape(1, HIDDEN), (NW, HIDDEN))])
  agg, sdp = _sc_aggregate(x, src, dst, ew, z128, z1)
  return _mlp_call(agg, sdp, sdm, x, W1, b1.reshape(1, HIDDEN),
                   W2, b2.reshape(1, HIDDEN))


# speedup vs baseline: 5.7338x; 1.0370x over previous
import jax
import jax.numpy as jnp
from jax import lax
from jax.experimental import pallas as pl
from jax.experimental.pallas import tpu as pltpu
from jax.experimental.pallas import tpu_sc as plsc

HIDDEN = 128
N_NODES = 10000
N_EDGES = 320000

NC = 2
NS = 16
NW = NC * NS
E_PER_TILE = N_EDGES // NW
CHUNK = 80
NCHUNKS = E_PER_TILE // CHUNK
SB = 5
ROWS_PER_TILE = N_NODES // NS
LANES = 16


def _sc_body(x_hbm, ei_hbm, ew_hbm, z128_hbm, z1_hbm,
             agg_out, sd_out,
             src_v, dst_v, ew_v, rows_v, s_part, d_part, acc_sh, gsems, ssems):
  c = lax.axis_index("c")
  s = lax.axis_index("s")
  wid = c * NS + s
  ebase = wid * E_PER_TILE

  rbase = s * ROWS_PER_TILE
  pltpu.sync_copy(z128_hbm.at[pl.ds(rbase, ROWS_PER_TILE)],
                  acc_sh.at[pl.ds(rbase, ROWS_PER_TILE)])
  pltpu.sync_copy(z1_hbm, s_part)
  pltpu.sync_copy(z1_hbm, d_part)
  plsc.subcore_barrier()

  ones16 = jnp.ones((LANES,), jnp.float32)
  cbase = wid * NCHUNKS

  def sb_body(i, carry):
    pltpu.sync_copy(ei_hbm.at[0, pl.ds(cbase + i * SB, SB)], src_v)
    pltpu.sync_copy(ei_hbm.at[1, pl.ds(cbase + i * SB, SB)], dst_v)
    pltpu.sync_copy(ew_hbm.at[pl.ds(cbase + i * SB, SB)], ew_v)
    sdesc = [None, None]
    for k in range(SB):
      b = k % 2
      if sdesc[b] is not None:
        sdesc[b].wait()
      gd = pltpu.async_copy(x_hbm.at[src_v.at[k]], rows_v.at[b], gsems.at[b])
      for g in range(CHUNK // LANES):
        idx = dst_v[k, pl.ds(g * LANES, LANES)]
        plsc.addupdate_scatter(s_part, [idx], ew_v[k, pl.ds(g * LANES, LANES)])
        plsc.addupdate_scatter(d_part, [idx], ones16)
      gd.wait()
      sdesc[b] = pltpu.async_copy(rows_v.at[b], acc_sh.at[dst_v.at[k]],
                                  ssems.at[b], add=True)
    sdesc[0].wait()
    sdesc[1].wait()
    return carry

  lax.fori_loop(0, NCHUNKS // SB, sb_body, 0)
  plsc.subcore_barrier()

  pltpu.sync_copy(acc_sh.at[pl.ds(rbase, ROWS_PER_TILE)],
                  agg_out.at[c, pl.ds(rbase, ROWS_PER_TILE)])
  pltpu.sync_copy(s_part, sd_out.at[wid])
  pltpu.sync_copy(d_part, sd_out.at[NW + wid])


_sc_aggregate = pl.kernel(
    _sc_body,
    out_type=(
        jax.ShapeDtypeStruct((NC, N_NODES, HIDDEN), jnp.float32),
        jax.ShapeDtypeStruct((2 * NW, N_NODES), jnp.float32),
    ),
    mesh=plsc.VectorSubcoreMesh(core_axis_name="c", subcore_axis_name="s",
                                num_cores=NC, num_subcores=NS),
    scratch_types=[
        pltpu.VMEM((SB, CHUNK), jnp.int32),
        pltpu.VMEM((SB, CHUNK), jnp.int32),
        pltpu.VMEM((SB, CHUNK), jnp.float32),
        pltpu.VMEM((2, CHUNK, HIDDEN), jnp.float32),
        pltpu.VMEM((N_NODES,), jnp.float32),
        pltpu.VMEM((N_NODES,), jnp.float32),
        pltpu.VMEM_SHARED((N_NODES, HIDDEN), jnp.float32),
        pltpu.SemaphoreType.DMA((2,)),
        pltpu.SemaphoreType.DMA((2,)),
    ],
    compiler_params=pltpu.CompilerParams(use_tc_tiling_on_sc=False,
                                         needs_layout_passes=False),
)


ROW_BLK = 1024


def _mlp_body(p0, p1, sdp, sdm, x, W1, b1, W2, b2, o):
  corr = lax.dot_general(sdp[...], sdm[...], (((0,), (0,)), ((), ())),
                         precision=lax.Precision.HIGHEST,
                         preferred_element_type=jnp.float32)
  pre = p0[...][0] + p1[...][0] + x[...] + corr
  h = jnp.maximum(
      jnp.dot(pre, W1[...], preferred_element_type=jnp.float32) + b1[...], 0.0)
  o[...] = jnp.dot(h, W2[...], preferred_element_type=jnp.float32) + b2[...]


def _mlp_call(agg, sdp, sdm, x, W1, b1, W2, b2):
  nblk = (N_NODES + ROW_BLK - 1) // ROW_BLK
  row = lambda i: (i, 0)
  fix = lambda i: (0, 0)
  return pl.pallas_call(
      _mlp_body,
      grid=(nblk,),
      in_specs=[
          pl.BlockSpec((1, ROW_BLK, HIDDEN), lambda i: (0, i, 0)),
          pl.BlockSpec((1, ROW_BLK, HIDDEN), lambda i: (1, i, 0)),
          pl.BlockSpec((2 * NW, ROW_BLK), lambda i: (0, i)),
          pl.BlockSpec((2 * NW, HIDDEN), fix),
          pl.BlockSpec((ROW_BLK, HIDDEN), row),
          pl.BlockSpec((HIDDEN, HIDDEN), fix),
          pl.BlockSpec((1, HIDDEN), fix),
          pl.BlockSpec((HIDDEN, HIDDEN), fix),
          pl.BlockSpec((1, HIDDEN), fix),
      ],
      out_specs=pl.BlockSpec((ROW_BLK, HIDDEN), row),
      out_shape=jax.ShapeDtypeStruct((N_NODES, HIDDEN), jnp.float32),
  )(agg, agg, sdp, sdm, x, W1, b1, W2, b2)


def kernel(x, edge_index, edge_weight, We, be, W1, b1, W2, b2):
  ei = edge_index.astype(jnp.int32).reshape(2, N_EDGES // CHUNK, CHUNK)
  ew = edge_weight.astype(jnp.float32).reshape(N_EDGES // CHUNK, CHUNK)
  z128 = jnp.zeros((N_NODES, HIDDEN), jnp.float32)
  z1 = jnp.zeros((N_NODES,), jnp.float32)
  sdm = jnp.concatenate([jnp.broadcast_to(We.reshape(1, HIDDEN), (NW, HIDDEN)),
                         jnp.broadcast_to(be.reshape(1, HIDDEN), (NW, HIDDEN))])
  agg, sdp = _sc_aggregate(x, ei, ew, z128, z1)
  return _mlp_call(agg, sdp, sdm, x, W1, b1.reshape(1, HIDDEN),
                   W2, b2.reshape(1, HIDDEN))
